# Initial kernel scaffold; baseline (speedup 1.0000x reference)
import jax
import jax.numpy as jnp
from jax.experimental import pallas as pl


def kernel(x, routing_token, gamma_light, w1, b1, w2, b2, gamma_heavy, w3, b3, w4, b4):
    raise NotImplementedError("write your pallas kernel here")


# speedup vs baseline: 1.7976x; 1.7976x over previous
import functools

import jax
import jax.numpy as jnp
from jax import lax
from jax.experimental import pallas as pl
from jax.experimental.pallas import tpu as pltpu

B = 2
N = 4096
DIM = 1024
NUM_HEAVY = 1024
LIGHT_H = 512
HEAVY_H = 4096
K_CD = 8
EPS_CD = 0.1
N_ITERS = 50

BT_L = 512
N_STEPS_L = (B * N) // BT_L
BT_H = 256
N_STEPS_H = (B * NUM_HEAVY) // BT_H


def _rmsnorm(x, gamma):
    norm = jnp.sqrt(jnp.sum(x * x, axis=-1, keepdims=True))
    normed = x / jnp.maximum(norm, 1e-12)
    return normed * (DIM ** 0.5) * gamma


def _gelu(x):
    return 0.5 * x * (1.0 + lax.erf(x * (2.0 ** -0.5)))


def _light_router_kernel(x_ref, rt_ref, gl_ref, w1_ref, b1_ref, w2_ref, b2_ref,
                         light_ref, scores_ref, s_acc):
    j = pl.program_id(0)
    xb = x_ref[...]
    s_part = jnp.dot(xb, rt_ref[...], preferred_element_type=jnp.float32)
    s_acc[pl.ds(j, 1), :] = s_part.reshape(1, BT_L)
    h = _rmsnorm(xb, gl_ref[...])
    h = jnp.dot(h, w1_ref[...], preferred_element_type=jnp.float32) + b1_ref[...]
    h = _gelu(h)
    light_ref[...] = (
        jnp.dot(h, w2_ref[...], preferred_element_type=jnp.float32) + b2_ref[...]
    )

    @pl.when(j == N_STEPS_L - 1)
    def _():
        s = s_acc[...].reshape(B, N)
        constant = EPS_CD * jnp.log(float(K_CD))
        b = -jnp.maximum(s, 0.0)
        a = jnp.zeros((B, 1), dtype=s.dtype)

        def body(_, carry):
            a, b = carry
            z = (s + b) / EPS_CD
            m = jnp.max(z, axis=-1, keepdims=True)
            lse = jnp.log(jnp.sum(jnp.exp(z - m), axis=-1, keepdims=True)) + m
            a = constant - EPS_CD * lse
            b = -jnp.maximum(s + a, 0.0)
            return a, b

        a, b = lax.fori_loop(0, N_ITERS, body, (a, b))
        scores_ref[...] = jnp.exp((s + a + b) / EPS_CD)


def _heavy_kernel(sel_ref, x_hbm, _light_alias, gh_ref, w3_ref, b3_ref, w4_ref,
                  b4_ref, out_hbm, xbuf, lbuf, obuf, sem_x, sem_l, sem_o):
    del _light_alias
    j = pl.program_id(0)
    base = j * BT_H

    def x_copy(i):
        gidx = sel_ref[base + i]
        return pltpu.make_async_copy(
            x_hbm.at[pl.ds(gidx, 1), :], xbuf.at[pl.ds(i, 1), :], sem_x)

    def l_copy(i):
        gidx = sel_ref[base + i]
        return pltpu.make_async_copy(
            out_hbm.at[pl.ds(gidx, 1), :], lbuf.at[pl.ds(i, 1), :], sem_l)

    def o_copy(i):
        gidx = sel_ref[base + i]
        return pltpu.make_async_copy(
            obuf.at[pl.ds(i, 1), :], out_hbm.at[pl.ds(gidx, 1), :], sem_o)

    lax.fori_loop(0, BT_H, lambda i, _: (x_copy(i).start(), 0)[1], 0)
    lax.fori_loop(0, BT_H, lambda i, _: (l_copy(i).start(), 0)[1], 0)
    lax.fori_loop(0, BT_H, lambda i, _: (x_copy(i).wait(), 0)[1], 0)

    xb = xbuf[...]
    h = _rmsnorm(xb, gh_ref[...])
    h = jnp.dot(h, w3_ref[...], preferred_element_type=jnp.float32) + b3_ref[...]
    h = _gelu(h)
    heavy = jnp.dot(h, w4_ref[...], preferred_element_type=jnp.float32) + b4_ref[...]

    lax.fori_loop(0, BT_H, lambda i, _: (l_copy(i).wait(), 0)[1], 0)
    obuf[...] = heavy + lbuf[...]
    lax.fori_loop(0, BT_H, lambda i, _: (o_copy(i).start(), 0)[1], 0)
    lax.fori_loop(0, BT_H, lambda i, _: (o_copy(i).wait(), 0)[1], 0)


def kernel(x, routing_token, gamma_light, w1, b1, w2, b2,
           gamma_heavy, w3, b3, w4, b4):
    xf = x.reshape(B * N, DIM)
    rt2 = routing_token.reshape(DIM, 1)
    gl = gamma_light.reshape(1, DIM)
    gh = gamma_heavy.reshape(1, DIM)
    b1r = b1.reshape(1, LIGHT_H)
    b2r = b2.reshape(1, DIM)
    b3r = b3.reshape(1, HEAVY_H)
    b4r = b4.reshape(1, DIM)

    lightf, scores = pl.pallas_call(
        _light_router_kernel,
        grid=(N_STEPS_L,),
        in_specs=[
            pl.BlockSpec((BT_L, DIM), lambda j: (j, 0)),
            pl.BlockSpec((DIM, 1), lambda j: (0, 0)),
            pl.BlockSpec((1, DIM), lambda j: (0, 0)),
            pl.BlockSpec((DIM, LIGHT_H), lambda j: (0, 0)),
            pl.BlockSpec((1, LIGHT_H), lambda j: (0, 0)),
            pl.BlockSpec((LIGHT_H, DIM), lambda j: (0, 0)),
            pl.BlockSpec((1, DIM), lambda j: (0, 0)),
        ],
        out_specs=[
            pl.BlockSpec((BT_L, DIM), lambda j: (j, 0)),
            pl.BlockSpec((B, N), lambda j: (0, 0)),
        ],
        out_shape=[
            jax.ShapeDtypeStruct((B * N, DIM), jnp.float32),
            jax.ShapeDtypeStruct((B, N), jnp.float32),
        ],
        scratch_shapes=[pltpu.VMEM((N_STEPS_L, BT_L), jnp.float32)],
        compiler_params=pltpu.CompilerParams(
            dimension_semantics=("arbitrary",),
        ),
    )(xf, rt2, gl, w1, b1r, w2, b2r)

    sel = lax.top_k(scores, NUM_HEAVY)[1].astype(jnp.int32)
    sel_flat = (sel + (jnp.arange(B, dtype=jnp.int32) * N)[:, None]).reshape(-1)

    outf = pl.pallas_call(
        _heavy_kernel,
        grid_spec=pltpu.PrefetchScalarGridSpec(
            num_scalar_prefetch=1,
            grid=(N_STEPS_H,),
            in_specs=[
                pl.BlockSpec(memory_space=pl.MemorySpace.ANY),
                pl.BlockSpec(memory_space=pl.MemorySpace.ANY),
                pl.BlockSpec((1, DIM), lambda j, sel: (0, 0)),
                pl.BlockSpec((DIM, HEAVY_H), lambda j, sel: (0, 0)),
                pl.BlockSpec((1, HEAVY_H), lambda j, sel: (0, 0)),
                pl.BlockSpec((HEAVY_H, DIM), lambda j, sel: (0, 0)),
                pl.BlockSpec((1, DIM), lambda j, sel: (0, 0)),
            ],
            out_specs=pl.BlockSpec(memory_space=pl.MemorySpace.ANY),
            scratch_shapes=[
                pltpu.VMEM((BT_H, DIM), jnp.float32),
                pltpu.VMEM((BT_H, DIM), jnp.float32),
                pltpu.VMEM((BT_H, DIM), jnp.float32),
                pltpu.SemaphoreType.DMA,
                pltpu.SemaphoreType.DMA,
                pltpu.SemaphoreType.DMA,
            ],
        ),
        out_shape=jax.ShapeDtypeStruct((B * N, DIM), jnp.float32),
        input_output_aliases={2: 0},
        compiler_params=pltpu.CompilerParams(
            dimension_semantics=("arbitrary",),
        ),
    )(sel_flat, xf, lightf, gh, w3, b3r, w4, b4r)

    return outf.reshape(B, N, DIM)
